# baseline (speedup 1.0000x reference)
import functools

import jax
import jax.numpy as jnp
from jax import lax
from jax.experimental import pallas as pl

HID = 128
N_OCC = 50000
N_CH = 5000
N_SEC = 2000
NUM_CLASSES = 1001
NUM_LAYERS = 2


def _ln(x, g, b):
    m = x.mean(-1, keepdims=True)
    v = ((x - m) ** 2).mean(-1, keepdims=True)
    return (x - m) / jnp.sqrt(v + 1e-5) * g + b


def _classifier_kernel(x_ref, w_ref, b_ref, o_ref):
    o_ref[...] = (
        lax.dot_general(x_ref[...], w_ref[...], (((1,), (1,)), ((), ())),
                        preferred_element_type=jnp.float32)
        + b_ref[...]
    )


def _classifier(h, w, b):
    blk = 1024
    grid = (pl.cdiv(N_OCC, blk),)
    return pl.pallas_call(
        _classifier_kernel,
        grid=grid,
        in_specs=[
            pl.BlockSpec((blk, HID), lambda i: (i, 0)),
            pl.BlockSpec((NUM_CLASSES, HID), lambda i: (0, 0)),
            pl.BlockSpec((1, NUM_CLASSES), lambda i: (0, 0)),
        ],
        out_specs=pl.BlockSpec((blk, NUM_CLASSES), lambda i: (i, 0)),
        out_shape=jax.ShapeDtypeStruct((N_OCC, NUM_CLASSES), jnp.float32),
    )(h, w, b.reshape(1, NUM_CLASSES))


def _seg_sum(vals, seg, n):
    return jax.ops.segment_sum(vals, seg, num_segments=n)


def kernel(x_occ, x_chord, x_sec, params, next_src, next_dst, prev_src,
           prev_dst, inst_src, inst_dst, insec_src, insec_dst, nsec_src,
           nsec_dst):
    Wo, bo = params["occ_proj"]
    Wc, bc = params["chord_proj"]
    Ws, bs = params["sec_proj"]

    ones = jnp.ones((50000,), jnp.float32)
    inv_n = 1.0 / jnp.clip(_seg_sum(ones, next_dst, N_OCC), 1.0)
    inv_p = 1.0 / jnp.clip(_seg_sum(ones, prev_dst, N_OCC), 1.0)
    inv_ir = 1.0 / jnp.clip(_seg_sum(ones, inst_src, N_OCC), 1.0)
    inv_sr = 1.0 / jnp.clip(_seg_sum(ones, insec_src, N_OCC), 1.0)
    inv_io = 1.0 / jnp.clip(_seg_sum(ones, inst_dst, N_CH), 1.0)
    inv_is = 1.0 / jnp.clip(_seg_sum(ones, insec_dst, N_SEC), 1.0)
    inv_ns = 1.0 / jnp.clip(
        _seg_sum(jnp.ones((2000,), jnp.float32), nsec_dst, N_SEC), 1.0)

    wn = 0.25 * inv_n[next_dst]
    wp = 0.25 * inv_p[prev_dst]
    wir = 0.25 * inv_ir[inst_src]
    wsr = 0.25 * inv_sr[insec_src]

    tcf = x_chord @ Wo[:, 64:].T
    cfh = jnp.zeros((N_OCC, HID), jnp.float32).at[inst_src].set(tcf[inst_dst])
    h_occ = x_occ @ Wo[:, :64].T + cfh + bo
    h_ch = x_chord @ Wc.T + bc
    h_sec = x_sec @ Ws.T + bs

    for l in range(NUM_LAYERS):
        p = params["layers"][l]
        g, b = params["norms"][l]
        Wn_l, bn, Wn_r = p["next"]
        Wp_l, bp, Wp_r = p["prev"]
        Wir_l, bir, Wir_r = p["inst_rev"]
        Wsr_l, bsr, Wsr_r = p["sec_rev"]
        Wio_l, bio, Wio_r = p["instance_of"]
        Wis_l, bis, Wis_r = p["in_section"]
        Wns_l, bns, Wns_r = p["next_section"]

        Tn = h_occ @ Wn_l.T
        Tp = h_occ @ Wp_l.T
        Tir = h_ch @ Wir_l.T
        Tsr = h_sec @ Wsr_l.T
        R = h_occ @ (Wn_r + Wp_r + Wir_r + Wsr_r).T
        Bsum = 0.25 * (bn + bp + bir + bsr)

        S = (
            _seg_sum(Tn[next_src] * wn[:, None], next_dst, N_OCC)
            + _seg_sum(Tp[prev_src] * wp[:, None], prev_dst, N_OCC)
            + _seg_sum(Tir[inst_dst] * wir[:, None], inst_src, N_OCC)
            + _seg_sum(Tsr[insec_dst] * wsr[:, None], insec_src, N_OCC)
        )
        o_occ = S + 0.25 * R + Bsum

        A_io = _seg_sum(h_occ[inst_src], inst_dst, N_CH) * inv_io[:, None]
        o_ch = A_io @ Wio_l.T + bio + h_ch @ Wio_r.T

        A_is = _seg_sum(h_occ[insec_src], insec_dst, N_SEC) * inv_is[:, None]
        A_ns = _seg_sum(h_sec[nsec_src], nsec_dst, N_SEC) * inv_ns[:, None]
        o_sec = 0.5 * (A_is @ Wis_l.T + bis + A_ns @ Wns_l.T + bns
                       + h_sec @ (Wis_r + Wns_r).T)

        h_occ = _ln(o_occ, g, b) + h_occ
        h_ch = _ln(o_ch, g, b) + h_ch
        h_sec = _ln(o_sec, g, b) + h_sec

    Wcl, bcl = params["classifier"]
    return _classifier(h_occ, Wcl, bcl)


# speedup vs baseline: 2.4077x; 2.4077x over previous
import functools

import jax
import jax.numpy as jnp
from jax import lax
from jax.experimental import pallas as pl
from jax.experimental.pallas import tpu as pltpu
from jax.experimental.pallas import tpu_sc as plsc

HID = 128
N_OCC = 50000
N_CH = 5000
N_SEC = 2000
NUM_CLASSES = 1001
NUM_LAYERS = 2

NP_OCC = 51200
NP_CH = 5120
NP_SEC = 2560
CHUNK = 12800
BUF_OCC = 13056
EPAD = 51200
EPAD_NS = 2048
W = 64

_mesh = plsc.VectorSubcoreMesh(core_axis_name="c", subcore_axis_name="s")


def _f32(shape):
    return jax.ShapeDtypeStruct(shape, jnp.float32)


def _counts_body(nd, pd, ird, srd, iod, isd, nsd,
                 c_n, c_p, c_ir, c_sr, c_io, c_is, c_ns,
                 acc, dstv, dloc, ones, zb):
    c = lax.axis_index("c")
    s = lax.axis_index("s")
    z16 = jnp.zeros((16,), jnp.float32)
    o16 = jnp.ones((16,), jnp.float32)
    for i in range(16):
        for j in range(8):
            zb[i, pl.ds(j * 16, 16)] = z16
    for i in range(W):
        for j in range(8):
            ones[i, pl.ds(j * 16, 16)] = o16

    def cagg(dst_hbm, out_hbm, per_tile, lo, out_lo, zero_rows, drain_rows,
             chunked):
        zpt = zero_rows // 16
        zbase = s * zpt
        for k in range(zpt // 16):
            pltpu.sync_copy(zb, acc.at[pl.ds(zbase + 16 * k, 16)])
        plsc.subcore_barrier()
        ebase = s * per_tile
        pltpu.sync_copy(dst_hbm.at[pl.ds(ebase, per_tile)],
                        dstv.at[pl.ds(0, per_tile)])

        @pl.loop(0, per_tile // W)
        def _(w):
            for j in range(W // 16):
                d = dstv[pl.ds(w * W + j * 16, 16)]
                if chunked:
                    inwin = (d >= lo) & (d < lo + CHUNK)
                    tr = CHUNK + ((w * W + j * 16 + lax.iota(jnp.int32, 16))
                                  & 255)
                    d = jnp.where(inwin, d - lo, tr)
                dloc[pl.ds(j * 16, 16)] = d
            pltpu.sync_copy(ones, acc.at[dloc], add=True)

        plsc.subcore_barrier()
        dpt = drain_rows // 16
        base = s * dpt
        pltpu.sync_copy(acc.at[pl.ds(base, dpt)],
                        out_hbm.at[pl.ds(out_lo + base, dpt)])
        plsc.subcore_barrier()

    ept = EPAD // 16

    @pl.when(c == 0)
    def _():
        cagg(iod, c_io, ept, 0, 0, NP_CH, NP_CH, False)

    @pl.when(c == 1)
    def _():
        cagg(isd, c_is, ept, 0, 0, NP_SEC, NP_SEC, False)
        cagg(nsd, c_ns, EPAD_NS // 16, 0, 0, NP_SEC, NP_SEC, False)

    for d_hbm, out in ((nd, c_n), (pd, c_p), (ird, c_ir), (srd, c_sr)):
        for p in range(2):
            lo = (c * 2 + p) * CHUNK
            cagg(d_hbm, out, ept, lo, lo, BUF_OCC, CHUNK, True)


def _sc_counts(nd, pd, ird, srd, iod, isd, nsd):
    f = pl.kernel(
        _counts_body,
        out_type=[_f32((NP_OCC, HID)), _f32((NP_OCC, HID)),
                  _f32((NP_OCC, HID)), _f32((NP_OCC, HID)),
                  _f32((NP_CH, HID)), _f32((NP_SEC, HID)),
                  _f32((NP_SEC, HID))],
        mesh=_mesh,
        scratch_types=[
            pltpu.VMEM_SHARED((BUF_OCC, HID), jnp.float32),
            pltpu.VMEM((EPAD // 16,), jnp.int32),
            pltpu.VMEM((W,), jnp.int32),
            pltpu.VMEM((W, HID), jnp.float32),
            pltpu.VMEM((16, HID), jnp.float32),
        ],
    )
    return f(nd, pd, ird, srd, iod, isd, nsd)


def _inv_kernel(scale, c_ref, o_ref):
    o_ref[...] = scale / jnp.maximum(c_ref[...][:, :16], 1.0)


def _inv(cnt, scale, blk):
    m = cnt.shape[0]
    return pl.pallas_call(
        functools.partial(_inv_kernel, scale),
        grid=(m // blk,),
        in_specs=[pl.BlockSpec((blk, HID), lambda i: (i, 0))],
        out_specs=pl.BlockSpec((blk, 16), lambda i: (i, 0)),
        out_shape=_f32((m, 16)),
    )(cnt)


def _layer_body(Tn, Tp, Tir, Tsr, h_occ, h_sec,
                n_s, n_d, p_s, p_d, ir_s, ir_d, sr_s, sr_d,
                io_s, io_d, is_s, is_d, ns_s, ns_d,
                A_n, A_p, A_ir, A_sr, A_io, A_is, A_ns,
                acc, srcv, dstv, bufA, bufB, dloc, zb, semA, semB):
    c = lax.axis_index("c")
    s = lax.axis_index("s")
    z16 = jnp.zeros((16,), jnp.float32)
    for i in range(16):
        for j in range(8):
            zb[i, pl.ds(j * 16, 16)] = z16

    def zero_acc(nrows):
        zpt = nrows // 16
        base = s * zpt
        for k in range(zpt // 16):
            pltpu.sync_copy(zb, acc.at[pl.ds(base + 16 * k, 16)])

    def start(table, w, buf, sem):
        pltpu.async_copy(table.at[srcv.at[pl.ds(w * W, W)]], buf, sem)

    def finish(table, w, buf, sem, lo, chunked):
        for j in range(W // 16):
            d = dstv[pl.ds(w * W + j * 16, 16)]
            if chunked:
                inwin = (d >= lo) & (d < lo + CHUNK)
                tr = CHUNK + ((w * W + j * 16 + lax.iota(jnp.int32, 16)) & 255)
                dl = jnp.where(inwin, d - lo, tr)
            else:
                dl = d
            dloc[pl.ds(j * 16, 16)] = dl
        pltpu.make_async_copy(table.at[srcv.at[pl.ds(w * W, W)]], buf, sem).wait()
        pltpu.sync_copy(buf, acc.at[dloc], add=True)

    def agg(table, src_hbm, dst_hbm, out_hbm, per_tile, lo, out_lo,
            zero_rows, drain_rows, chunked):
        zero_acc(zero_rows)
        plsc.subcore_barrier()
        ebase = s * per_tile
        pltpu.sync_copy(src_hbm.at[pl.ds(ebase, per_tile)],
                        srcv.at[pl.ds(0, per_tile)])
        pltpu.sync_copy(dst_hbm.at[pl.ds(ebase, per_tile)],
                        dstv.at[pl.ds(0, per_tile)])
        nwin = per_tile // W
        assert nwin % 2 == 0
        start(table, 0, bufA, semA)
        start(table, 1, bufB, semB)

        @pl.loop(0, nwin, step=2)
        def _(w):
            finish(table, w, bufA, semA, lo, chunked)

            @pl.when(w + 2 < nwin)
            def _():
                start(table, w + 2, bufA, semA)

            finish(table, w + 1, bufB, semB, lo, chunked)

            @pl.when(w + 3 < nwin)
            def _():
                start(table, w + 3, bufB, semB)
        plsc.subcore_barrier()
        dpt = drain_rows // 16
        base = s * dpt
        pltpu.sync_copy(acc.at[pl.ds(base, dpt)],
                        out_hbm.at[pl.ds(out_lo + base, dpt)])
        plsc.subcore_barrier()

    ept = EPAD // 16

    @pl.when(c == 0)
    def _():
        agg(h_occ, io_s, io_d, A_io, ept, 0, 0, NP_CH, NP_CH, False)

    @pl.when(c == 1)
    def _():
        agg(h_occ, is_s, is_d, A_is, ept, 0, 0, NP_SEC, NP_SEC, False)
        agg(h_sec, ns_s, ns_d, A_ns, EPAD_NS // 16, 0, 0, NP_SEC, NP_SEC, False)

    for table, sp, dp, out in ((Tn, n_s, n_d, A_n), (Tp, p_s, p_d, A_p),
                               (Tir, ir_s, ir_d, A_ir), (Tsr, sr_s, sr_d, A_sr)):
        for p in range(2):
            lo = (c * 2 + p) * CHUNK
            agg(table, sp, dp, out, ept, lo, lo, BUF_OCC, CHUNK, True)


def _sc_layer(Tn, Tp, Tir, Tsr, h_occ, h_sec, idx):
    f = pl.kernel(
        _layer_body,
        out_type=[_f32((NP_OCC, HID)), _f32((NP_OCC, HID)), _f32((NP_OCC, HID)),
                  _f32((NP_OCC, HID)), _f32((NP_CH, HID)), _f32((NP_SEC, HID)),
                  _f32((NP_SEC, HID))],
        mesh=_mesh,
        scratch_types=[
            pltpu.VMEM_SHARED((BUF_OCC, HID), jnp.float32),
            pltpu.VMEM((EPAD // 16,), jnp.int32),
            pltpu.VMEM((EPAD // 16,), jnp.int32),
            pltpu.VMEM((W, HID), jnp.float32),
            pltpu.VMEM((W, HID), jnp.float32),
            pltpu.VMEM((W,), jnp.int32),
            pltpu.VMEM((16, HID), jnp.float32),
            pltpu.SemaphoreType.DMA,
            pltpu.SemaphoreType.DMA,
        ],
    )
    return f(Tn, Tp, Tir, Tsr, h_occ, h_sec, *idx)


def _mm_kernel(nout, addc, x_ref, *refs):
    w_refs = refs[:nout]
    b_refs = refs[nout:2 * nout]
    k = 2 * nout
    c_ref = None
    if addc:
        c_ref = refs[k]
        k += 1
    out_refs = refs[k:]
    x = x_ref[...]
    for i in range(nout):
        o = jnp.dot(x, w_refs[i][...], preferred_element_type=jnp.float32)
        o = o + b_refs[i][...]
        if addc and i == 0:
            o = o + c_ref[...]
        out_refs[i][...] = o


def _mm(x, wts, biases, add=None, blk=1024):
    m = x.shape[0]
    assert m % blk == 0
    nout = len(wts)
    grid = (m // blk,)
    in_specs = [pl.BlockSpec((blk, x.shape[1]), lambda i: (i, 0))]
    args = [x]
    for wt in wts:
        in_specs.append(pl.BlockSpec(wt.shape, lambda i: (0, 0)))
        args.append(wt)
    for b in biases:
        in_specs.append(pl.BlockSpec((1, b.shape[0]), lambda i: (0, 0)))
        args.append(b.reshape(1, -1))
    if add is not None:
        in_specs.append(pl.BlockSpec((blk, add.shape[1]), lambda i: (i, 0)))
        args.append(add)
    outs = pl.pallas_call(
        functools.partial(_mm_kernel, nout, add is not None),
        grid=grid,
        in_specs=in_specs,
        out_specs=[pl.BlockSpec((blk, wt.shape[1]), lambda i: (i, 0))
                   for wt in wts],
        out_shape=[_f32((m, wt.shape[1])) for wt in wts],
    )(*args)
    return outs


def _ln_block(o, g, b):
    mu = o.mean(-1, keepdims=True)
    d = o - mu
    v = (d * d).mean(-1, keepdims=True)
    return d * lax.rsqrt(v + 1e-5) * g + b


def _merge_occ_kernel(an, ap, air, asr, cn, cp, cir, csr, r, h, g, b, bs, out):
    o = r[...] + bs[...]
    for a, cc in ((an, cn), (ap, cp), (air, cir), (asr, csr)):
        o = o + a[...] * cc[...][:, :1]
    out[...] = _ln_block(o, g[...], b[...]) + h[...]


def _merge_occ(A4, C4, r, h, g, b, bs, blk=1024):
    m = h.shape[0]
    grid = (m // blk,)
    specA = pl.BlockSpec((blk, HID), lambda i: (i, 0))
    specC = pl.BlockSpec((blk, 16), lambda i: (i, 0))
    specV = pl.BlockSpec((1, HID), lambda i: (0, 0))
    return pl.pallas_call(
        _merge_occ_kernel,
        grid=grid,
        in_specs=[specA] * 4 + [specC] * 4 + [specA, specA, specV, specV, specV],
        out_specs=specA,
        out_shape=_f32((m, HID)),
    )(*A4, *C4, r, h, g.reshape(1, HID), b.reshape(1, HID), bs.reshape(1, HID))


def _merge_small_kernel(nagg, refs):
    k = 0
    aggs = []
    for _ in range(nagg):
        aggs.append((refs[k], refs[k + 1]))
        k += 2
    h = refs[k][...]; k += 1
    wls = [refs[k + i] for i in range(nagg)]; k += nagg
    wr = refs[k]; k += 1
    bias = refs[k]; k += 1
    g = refs[k]; k += 1
    b = refs[k]; k += 1
    out = refs[k]
    o = jnp.dot(h, wr[...], preferred_element_type=jnp.float32) + bias[...]
    for i, (a, cc) in enumerate(aggs):
        o = o + jnp.dot(a[...] * cc[...][:, :1], wls[i][...],
                        preferred_element_type=jnp.float32)
    out[...] = _ln_block(o, g[...], b[...]) + h


def _merge_small(aggs, cnts, h, wls, wr, bias, g, b, blk):
    m = h.shape[0]
    nagg = len(aggs)
    grid = (m // blk,)
    specA = pl.BlockSpec((blk, HID), lambda i: (i, 0))
    specC = pl.BlockSpec((blk, 16), lambda i: (i, 0))
    specW = pl.BlockSpec((HID, HID), lambda i: (0, 0))
    specV = pl.BlockSpec((1, HID), lambda i: (0, 0))
    in_specs = []
    args = []
    for a, cc in zip(aggs, cnts):
        in_specs += [specA, specC]
        args += [a, cc]
    in_specs.append(specA); args.append(h)
    for wl in wls:
        in_specs.append(specW); args.append(wl)
    in_specs += [specW, specV, specV, specV]
    args += [wr, bias.reshape(1, HID), g.reshape(1, HID), b.reshape(1, HID)]

    def body(*refs):
        _merge_small_kernel(nagg, refs)

    return pl.pallas_call(
        body, grid=grid, in_specs=in_specs, out_specs=specA,
        out_shape=_f32((m, HID)),
    )(*args)


def _classifier_kernel(x_ref, w_ref, b_ref, o_ref):
    o_ref[...] = (jnp.dot(x_ref[...], w_ref[...],
                          preferred_element_type=jnp.float32) + b_ref[...])


def _classifier(h_pad, wt, b):
    blk = 1000
    grid = (N_OCC // blk,)
    return pl.pallas_call(
        _classifier_kernel,
        grid=grid,
        in_specs=[
            pl.BlockSpec((blk, HID), lambda i: (i, 0)),
            pl.BlockSpec((HID, NUM_CLASSES), lambda i: (0, 0)),
            pl.BlockSpec((1, NUM_CLASSES), lambda i: (0, 0)),
        ],
        out_specs=pl.BlockSpec((blk, NUM_CLASSES), lambda i: (i, 0)),
        out_shape=_f32((N_OCC, NUM_CLASSES)),
    )(h_pad, wt, b.reshape(1, NUM_CLASSES))


def _pad_rows(x, n):
    return jnp.pad(x, ((0, n - x.shape[0]), (0, 0)))


def _pad_idx(idx, n, mode, npad_dst=0):
    e = idx.shape[0]
    i = jnp.arange(n - e, dtype=jnp.int32)
    if mode == "src":
        fill = (i * 97) % jnp.int32(1024)
    else:
        fill = npad_dst + (i % jnp.int32(1024 if npad_dst == N_OCC else 120))
    return jnp.concatenate([idx.astype(jnp.int32), fill])


def kernel(x_occ, x_chord, x_sec, params, next_src, next_dst, prev_src,
           prev_dst, inst_src, inst_dst, insec_src, insec_dst, nsec_src,
           nsec_dst):
    Wo, bo = params["occ_proj"]
    Wc, bc = params["chord_proj"]
    Ws, bs = params["sec_proj"]

    xo = _pad_rows(x_occ, NP_OCC)
    xc = _pad_rows(x_chord, NP_CH)
    xs = _pad_rows(x_sec, NP_SEC)

    n_s = _pad_idx(next_src, EPAD, "src")
    n_d = _pad_idx(next_dst, EPAD, "dst", N_OCC)
    p_s = _pad_idx(prev_src, EPAD, "src")
    p_d = _pad_idx(prev_dst, EPAD, "dst", N_OCC)
    ir_s = _pad_idx(inst_dst, EPAD, "src")
    ir_d = _pad_idx(inst_src, EPAD, "dst", N_OCC)
    sr_s = _pad_idx(insec_dst, EPAD, "src")
    sr_d = _pad_idx(insec_src, EPAD, "dst", N_OCC)
    io_s = _pad_idx(inst_src, EPAD, "src")
    io_d = _pad_idx(inst_dst, EPAD, "dst", N_CH)
    is_s = _pad_idx(insec_src, EPAD, "src")
    is_d = _pad_idx(insec_dst, EPAD, "dst", N_SEC)
    ns_s = _pad_idx(nsec_src, EPAD_NS, "src")
    ns_d = _pad_idx(nsec_dst, EPAD_NS, "dst", N_SEC)
    idx = (n_s, n_d, p_s, p_d, ir_s, ir_d, sr_s, sr_d,
           io_s, io_d, is_s, is_d, ns_s, ns_d)

    c_n, c_p, c_ir, c_sr, c_io, c_is, c_ns = _sc_counts(
        n_d, p_d, ir_d, sr_d, io_d, is_d, ns_d)
    c_n = _inv(c_n, 0.25, 1024)
    c_p = _inv(c_p, 0.25, 1024)
    c_ir = _inv(c_ir, 0.25, 1024)
    c_sr = _inv(c_sr, 0.25, 1024)
    c_io = _inv(c_io, 1.0, 1024)
    c_is = _inv(c_is, 1.0, 1280)
    c_ns = _inv(c_ns, 1.0, 1280)

    tcf, = _mm(xc, [Wo[:, 64:].T], [jnp.zeros((HID,), jnp.float32)], blk=1024)
    cfh = jnp.zeros((NP_OCC, HID), jnp.float32).at[inst_src].set(tcf[inst_dst])
    h_occ, = _mm(xo, [Wo[:, :64].T], [bo], add=cfh, blk=1024)
    h_ch, = _mm(xc, [Wc.T], [bc], blk=1024)
    h_sec, = _mm(xs, [Ws.T], [bs], blk=1280)

    zb = jnp.zeros((HID,), jnp.float32)
    for l in range(NUM_LAYERS):
        p = params["layers"][l]
        g, b = params["norms"][l]
        Wn_l, bn, Wn_r = p["next"]
        Wp_l, bp, Wp_r = p["prev"]
        Wir_l, bir, Wir_r = p["inst_rev"]
        Wsr_l, bsr, Wsr_r = p["sec_rev"]
        Wio_l, bio, Wio_r = p["instance_of"]
        Wis_l, bis, Wis_r = p["in_section"]
        Wns_l, bns, Wns_r = p["next_section"]

        wr_sum = 0.25 * (Wn_r + Wp_r + Wir_r + Wsr_r).T
        bsum = 0.25 * (bn + bp + bir + bsr)

        Tn, Tp, R = _mm(h_occ, [Wn_l.T, Wp_l.T, wr_sum], [zb, zb, bsum],
                        blk=1024)
        Tir, = _mm(h_ch, [Wir_l.T], [zb], blk=1024)
        Tsr, = _mm(h_sec, [Wsr_l.T], [zb], blk=1280)

        A_n, A_p, A_ir, A_sr, A_io, A_is, A_ns = _sc_layer(
            Tn, Tp, Tir, Tsr, h_occ, h_sec, idx)

        h_occ = _merge_occ((A_n, A_p, A_ir, A_sr), (c_n, c_p, c_ir, c_sr),
                           R, h_occ, g, b, jnp.zeros((HID,), jnp.float32))
        h_ch = _merge_small([A_io], [c_io], h_ch, [Wio_l.T], Wio_r.T, bio,
                            g, b, blk=1024)
        h_sec = _merge_small([A_is, A_ns], [c_is, c_ns], h_sec,
                             [0.5 * Wis_l.T, 0.5 * Wns_l.T],
                             0.5 * (Wis_r + Wns_r).T, 0.5 * (bis + bns),
                             g, b, blk=1280)

    Wcl, bcl = params["classifier"]
    return _classifier(h_occ, Wcl.T, bcl)


# speedup vs baseline: 2.6380x; 1.0957x over previous
import dataclasses
import functools

import jax
import jax.numpy as jnp
from jax import lax
from jax.experimental import pallas as pl
from jax.experimental.pallas import tpu as pltpu
from jax.experimental.pallas import tpu_sc as plsc

HID = 128
N_OCC = 50000
N_CH = 5000
N_SEC = 2000
NUM_CLASSES = 1001
NUM_LAYERS = 2

NP_OCC = 51200
NP_CH = 5120
NP_SEC = 2560
CHUNK = 12800
BUF_OCC = 13056
EPAD = 51200
EPAD_NS = 2048
W = 64

_mesh = plsc.VectorSubcoreMesh(core_axis_name="c", subcore_axis_name="s")


def _f32(shape):
    return jax.ShapeDtypeStruct(shape, jnp.float32)


def _counts_body(nd, pd, ird, srd, iod, isd, nsd,
                 c_n, c_p, c_ir, c_sr, c_io, c_is, c_ns,
                 acc, dstv, dloc, ones, zb):
    c = lax.axis_index("c")
    s = lax.axis_index("s")
    z16 = jnp.zeros((16,), jnp.float32)
    o16 = jnp.ones((16,), jnp.float32)
    for i in range(16):
        for j in range(8):
            zb[i, pl.ds(j * 16, 16)] = z16
    for i in range(W):
        for j in range(8):
            ones[i, pl.ds(j * 16, 16)] = o16

    def cagg(dst_hbm, out_hbm, per_tile, lo, out_lo, zero_rows, drain_rows,
             chunked):
        zpt = zero_rows // 16
        zbase = s * zpt
        for k in range(zpt // 16):
            pltpu.sync_copy(zb, acc.at[pl.ds(zbase + 16 * k, 16)])
        plsc.subcore_barrier()
        ebase = s * per_tile
        pltpu.sync_copy(dst_hbm.at[pl.ds(ebase, per_tile)],
                        dstv.at[pl.ds(0, per_tile)])

        @pl.loop(0, per_tile // W)
        def _(w):
            for j in range(W // 16):
                d = dstv[pl.ds(w * W + j * 16, 16)]
                if chunked:
                    inwin = (d >= lo) & (d < lo + CHUNK)
                    tr = CHUNK + ((w * W + j * 16 + lax.iota(jnp.int32, 16))
                                  & 255)
                    d = jnp.where(inwin, d - lo, tr)
                dloc[pl.ds(j * 16, 16)] = d
            pltpu.sync_copy(ones, acc.at[dloc], add=True)

        plsc.subcore_barrier()
        dpt = drain_rows // 16
        base = s * dpt
        pltpu.sync_copy(acc.at[pl.ds(base, dpt)],
                        out_hbm.at[pl.ds(out_lo + base, dpt)])
        plsc.subcore_barrier()

    ept = EPAD // 16

    @pl.when(c == 0)
    def _():
        cagg(iod, c_io, ept, 0, 0, NP_CH, NP_CH, False)

    @pl.when(c == 1)
    def _():
        cagg(isd, c_is, ept, 0, 0, NP_SEC, NP_SEC, False)
        cagg(nsd, c_ns, EPAD_NS // 16, 0, 0, NP_SEC, NP_SEC, False)

    for d_hbm, out in ((nd, c_n), (pd, c_p), (ird, c_ir), (srd, c_sr)):
        for p in range(2):
            lo = (c * 2 + p) * CHUNK
            cagg(d_hbm, out, ept, lo, lo, BUF_OCC, CHUNK, True)


def _sc_counts(nd, pd, ird, srd, iod, isd, nsd):
    f = pl.kernel(
        _counts_body,
        out_type=[_f32((NP_OCC, HID)), _f32((NP_OCC, HID)),
                  _f32((NP_OCC, HID)), _f32((NP_OCC, HID)),
                  _f32((NP_CH, HID)), _f32((NP_SEC, HID)),
                  _f32((NP_SEC, HID))],
        mesh=_mesh,
        scratch_types=[
            pltpu.VMEM_SHARED((BUF_OCC, HID), jnp.float32),
            pltpu.VMEM((EPAD // 16,), jnp.int32),
            pltpu.VMEM((W,), jnp.int32),
            pltpu.VMEM((W, HID), jnp.float32),
            pltpu.VMEM((16, HID), jnp.float32),
        ],
    )
    return f(nd, pd, ird, srd, iod, isd, nsd)


def _cf_win_body(ird, out, partial, ev, neg):
    c = lax.axis_index("c")
    s = lax.axis_index("s")

    @pl.when(c == 0)
    def _():
        n16 = jnp.full((16,), -1, jnp.int32)
        for j in range(8):
            neg[pl.ds(j * 16, 16)] = n16

        @pl.loop(0, NP_OCC // 16)
        def _(i):
            partial[pl.ds(i * 16, 16)] = neg[pl.ds(0, 16)]

        ept = EPAD // 16
        ebase = s * ept
        pltpu.sync_copy(ird.at[pl.ds(ebase, ept)], ev)
        iota = lax.iota(jnp.int32, 16)

        @pl.loop(0, ept // 16)
        def _(i):
            d = ev[pl.ds(i * 16, 16)]
            k2 = d * 32 + iota
            e = ebase + i * 16 + iota
            k2s, es = plsc.sort_key_val(k2, e)
            ds_ = lax.shift_right_logical(k2s, 5)
            dn = lax.gather(
                ds_, jnp.minimum(iota + 1, 15).reshape(16, 1),
                lax.GatherDimensionNumbers(offset_dims=(),
                                           collapsed_slice_dims=(0,),
                                           start_index_map=(0,)),
                (1,), mode=lax.GatherScatterMode.PROMISE_IN_BOUNDS)
            wm = (iota == 15) | (ds_ != dn)
            plsc.store_scatter(partial, [ds_], es, mask=wm)

        pltpu.sync_copy(partial, out.at[s])


def _cf_gather_body(wins, iod, tcf, out, pbuf, wbuf, iodv, idx2,
                    bufA, bufB, semA, semB):
    c = lax.axis_index("c")
    s = lax.axis_index("s")

    @pl.when(c == 0)
    def _():
        spt = NP_OCC // 16
        sbase = s * spt
        pltpu.sync_copy(iod, iodv)
        pltpu.sync_copy(wins.at[0].at[pl.ds(sbase, spt)], wbuf)
        for t in range(1, 16):
            pltpu.sync_copy(wins.at[t].at[pl.ds(sbase, spt)], pbuf)

            @pl.loop(0, spt // 16)
            def _(i):
                wbuf[pl.ds(i * 16, 16)] = jnp.maximum(
                    wbuf[pl.ds(i * 16, 16)], pbuf[pl.ds(i * 16, 16)])
        iota = lax.iota(jnp.int32, 16)

        @pl.loop(0, spt // 16)
        def _(i):
            wv = wbuf[pl.ds(i * 16, 16)]
            safe = jnp.maximum(wv, 0)
            raw = plsc.load_gather(iodv, [safe])
            tr = N_CH + ((i * 16 + iota) % 120)
            idx2[pl.ds(i * 16, 16)] = jnp.where(wv >= 0, raw, tr)

        def start(w, buf, sem):
            pltpu.async_copy(tcf.at[idx2.at[pl.ds(w * W, W)]], buf, sem)

        def finish(w, buf, sem):
            pltpu.make_async_copy(tcf.at[idx2.at[pl.ds(w * W, W)]], buf,
                                  sem).wait()
            pltpu.sync_copy(buf, out.at[pl.ds(sbase + w * W, W)])

        nwin = spt // W
        start(0, bufA, semA)
        start(1, bufB, semB)

        @pl.loop(0, nwin, step=2)
        def _(w):
            finish(w, bufA, semA)

            @pl.when(w + 2 < nwin)
            def _():
                start(w + 2, bufA, semA)

            finish(w + 1, bufB, semB)

            @pl.when(w + 3 < nwin)
            def _():
                start(w + 3, bufB, semB)


_CP_NO_LAYOUT = pltpu.CompilerParams()
if "needs_layout_passes" in pltpu.CompilerParams.__dataclass_fields__:
    _CP_NO_LAYOUT = dataclasses.replace(_CP_NO_LAYOUT,
                                        needs_layout_passes=False)


def _sc_cf(ir_d, io_d, tcf):
    ept = EPAD // 16
    spt = NP_OCC // 16
    win = pl.kernel(
        _cf_win_body,
        out_type=[jax.ShapeDtypeStruct((16, NP_OCC), jnp.int32)],
        mesh=_mesh,
        compiler_params=_CP_NO_LAYOUT,
        scratch_types=[
            pltpu.VMEM((NP_OCC,), jnp.int32),
            pltpu.VMEM((ept,), jnp.int32),
            pltpu.VMEM((128,), jnp.int32),
        ],
    )(ir_d)[0]
    cfh = pl.kernel(
        _cf_gather_body,
        out_type=[_f32((NP_OCC, HID))],
        mesh=_mesh,
        compiler_params=_CP_NO_LAYOUT,
        scratch_types=[
            pltpu.VMEM((spt,), jnp.int32),
            pltpu.VMEM((spt,), jnp.int32),
            pltpu.VMEM((EPAD,), jnp.int32),
            pltpu.VMEM((spt,), jnp.int32),
            pltpu.VMEM((W, HID), jnp.float32),
            pltpu.VMEM((W, HID), jnp.float32),
            pltpu.SemaphoreType.DMA,
            pltpu.SemaphoreType.DMA,
        ],
    )(win, io_d, tcf)[0]
    return cfh


def _inv_kernel(scale, c_ref, o_ref):
    o_ref[...] = scale / jnp.maximum(c_ref[...][:, :16], 1.0)


def _inv(cnt, scale, blk):
    m = cnt.shape[0]
    return pl.pallas_call(
        functools.partial(_inv_kernel, scale),
        grid=(m // blk,),
        in_specs=[pl.BlockSpec((blk, HID), lambda i: (i, 0))],
        out_specs=pl.BlockSpec((blk, 16), lambda i: (i, 0)),
        out_shape=_f32((m, 16)),
    )(cnt)


def _layer_body(Tn, Tp, Tir, Tsr, h_occ, h_sec,
                n_s, n_d, p_s, p_d, ir_s, ir_d, sr_s, sr_d,
                io_s, io_d, is_s, is_d, ns_s, ns_d,
                A_n, A_p, A_ir, A_sr, A_io, A_is, A_ns,
                acc, srcv, dstv, bufA, bufB, dloc, zb, semA, semB):
    c = lax.axis_index("c")
    s = lax.axis_index("s")
    z16 = jnp.zeros((16,), jnp.float32)
    for i in range(16):
        for j in range(8):
            zb[i, pl.ds(j * 16, 16)] = z16

    def zero_acc(nrows):
        zpt = nrows // 16
        base = s * zpt
        for k in range(zpt // 16):
            pltpu.sync_copy(zb, acc.at[pl.ds(base + 16 * k, 16)])

    def start(table, w, buf, sem):
        pltpu.async_copy(table.at[srcv.at[pl.ds(w * W, W)]], buf, sem)

    def finish(table, w, buf, sem, lo, chunked):
        for j in range(W // 16):
            d = dstv[pl.ds(w * W + j * 16, 16)]
            if chunked:
                inwin = (d >= lo) & (d < lo + CHUNK)
                tr = CHUNK + ((w * W + j * 16 + lax.iota(jnp.int32, 16)) & 255)
                dl = jnp.where(inwin, d - lo, tr)
            else:
                dl = d
            dloc[pl.ds(j * 16, 16)] = dl
        pltpu.make_async_copy(table.at[srcv.at[pl.ds(w * W, W)]], buf, sem).wait()
        pltpu.sync_copy(buf, acc.at[dloc], add=True)

    def agg(table, src_hbm, dst_hbm, out_hbm, per_tile, lo, out_lo,
            zero_rows, drain_rows, chunked):
        zero_acc(zero_rows)
        plsc.subcore_barrier()
        ebase = s * per_tile
        pltpu.sync_copy(src_hbm.at[pl.ds(ebase, per_tile)],
                        srcv.at[pl.ds(0, per_tile)])
        pltpu.sync_copy(dst_hbm.at[pl.ds(ebase, per_tile)],
                        dstv.at[pl.ds(0, per_tile)])
        nwin = per_tile // W
        assert nwin % 2 == 0
        start(table, 0, bufA, semA)
        start(table, 1, bufB, semB)

        @pl.loop(0, nwin, step=2)
        def _(w):
            finish(table, w, bufA, semA, lo, chunked)

            @pl.when(w + 2 < nwin)
            def _():
                start(table, w + 2, bufA, semA)

            finish(table, w + 1, bufB, semB, lo, chunked)

            @pl.when(w + 3 < nwin)
            def _():
                start(table, w + 3, bufB, semB)
        plsc.subcore_barrier()
        dpt = drain_rows // 16
        base = s * dpt
        pltpu.sync_copy(acc.at[pl.ds(base, dpt)],
                        out_hbm.at[pl.ds(out_lo + base, dpt)])
        plsc.subcore_barrier()

    ept = EPAD // 16

    @pl.when(c == 0)
    def _():
        agg(h_occ, io_s, io_d, A_io, ept, 0, 0, NP_CH, NP_CH, False)

    @pl.when(c == 1)
    def _():
        agg(h_occ, is_s, is_d, A_is, ept, 0, 0, NP_SEC, NP_SEC, False)
        agg(h_sec, ns_s, ns_d, A_ns, EPAD_NS // 16, 0, 0, NP_SEC, NP_SEC, False)

    for table, sp, dp, out in ((Tn, n_s, n_d, A_n), (Tp, p_s, p_d, A_p),
                               (Tir, ir_s, ir_d, A_ir), (Tsr, sr_s, sr_d, A_sr)):
        for p in range(2):
            lo = (c * 2 + p) * CHUNK
            agg(table, sp, dp, out, ept, lo, lo, BUF_OCC, CHUNK, True)


def _sc_layer(Tn, Tp, Tir, Tsr, h_occ, h_sec, idx):
    f = pl.kernel(
        _layer_body,
        out_type=[_f32((NP_OCC, HID)), _f32((NP_OCC, HID)), _f32((NP_OCC, HID)),
                  _f32((NP_OCC, HID)), _f32((NP_CH, HID)), _f32((NP_SEC, HID)),
                  _f32((NP_SEC, HID))],
        mesh=_mesh,
        scratch_types=[
            pltpu.VMEM_SHARED((BUF_OCC, HID), jnp.float32),
            pltpu.VMEM((EPAD // 16,), jnp.int32),
            pltpu.VMEM((EPAD // 16,), jnp.int32),
            pltpu.VMEM((W, HID), jnp.float32),
            pltpu.VMEM((W, HID), jnp.float32),
            pltpu.VMEM((W,), jnp.int32),
            pltpu.VMEM((16, HID), jnp.float32),
            pltpu.SemaphoreType.DMA,
            pltpu.SemaphoreType.DMA,
        ],
    )
    return f(Tn, Tp, Tir, Tsr, h_occ, h_sec, *idx)


def _mm_kernel(nout, addc, x_ref, *refs):
    w_refs = refs[:nout]
    b_refs = refs[nout:2 * nout]
    k = 2 * nout
    c_ref = None
    if addc:
        c_ref = refs[k]
        k += 1
    out_refs = refs[k:]
    x = x_ref[...]
    for i in range(nout):
        o = jnp.dot(x, w_refs[i][...], preferred_element_type=jnp.float32)
        o = o + b_refs[i][...]
        if addc and i == 0:
            o = o + c_ref[...]
        out_refs[i][...] = o


def _mm(x, wts, biases, add=None, blk=1024):
    m = x.shape[0]
    assert m % blk == 0
    nout = len(wts)
    grid = (m // blk,)
    in_specs = [pl.BlockSpec((blk, x.shape[1]), lambda i: (i, 0))]
    args = [x]
    for wt in wts:
        in_specs.append(pl.BlockSpec(wt.shape, lambda i: (0, 0)))
        args.append(wt)
    for b in biases:
        in_specs.append(pl.BlockSpec((1, b.shape[0]), lambda i: (0, 0)))
        args.append(b.reshape(1, -1))
    if add is not None:
        in_specs.append(pl.BlockSpec((blk, add.shape[1]), lambda i: (i, 0)))
        args.append(add)
    outs = pl.pallas_call(
        functools.partial(_mm_kernel, nout, add is not None),
        grid=grid,
        in_specs=in_specs,
        out_specs=[pl.BlockSpec((blk, wt.shape[1]), lambda i: (i, 0))
                   for wt in wts],
        out_shape=[_f32((m, wt.shape[1])) for wt in wts],
    )(*args)
    return outs


def _ln_block(o, g, b):
    mu = o.mean(-1, keepdims=True)
    d = o - mu
    v = (d * d).mean(-1, keepdims=True)
    return d * lax.rsqrt(v + 1e-5) * g + b


def _merge_occ_kernel(an, ap, air, asr, cn, cp, cir, csr, r, h, g, b, bs, out):
    o = r[...] + bs[...]
    for a, cc in ((an, cn), (ap, cp), (air, cir), (asr, csr)):
        o = o + a[...] * cc[...][:, :1]
    out[...] = _ln_block(o, g[...], b[...]) + h[...]


def _merge_occ(A4, C4, r, h, g, b, bs, blk=1024):
    m = h.shape[0]
    grid = (m // blk,)
    specA = pl.BlockSpec((blk, HID), lambda i: (i, 0))
    specC = pl.BlockSpec((blk, 16), lambda i: (i, 0))
    specV = pl.BlockSpec((1, HID), lambda i: (0, 0))
    return pl.pallas_call(
        _merge_occ_kernel,
        grid=grid,
        in_specs=[specA] * 4 + [specC] * 4 + [specA, specA, specV, specV, specV],
        out_specs=specA,
        out_shape=_f32((m, HID)),
    )(*A4, *C4, r, h, g.reshape(1, HID), b.reshape(1, HID), bs.reshape(1, HID))


def _merge_small_kernel(nagg, refs):
    k = 0
    aggs = []
    for _ in range(nagg):
        aggs.append((refs[k], refs[k + 1]))
        k += 2
    h = refs[k][...]; k += 1
    wls = [refs[k + i] for i in range(nagg)]; k += nagg
    wr = refs[k]; k += 1
    bias = refs[k]; k += 1
    g = refs[k]; k += 1
    b = refs[k]; k += 1
    out = refs[k]
    o = jnp.dot(h, wr[...], preferred_element_type=jnp.float32) + bias[...]
    for i, (a, cc) in enumerate(aggs):
        o = o + jnp.dot(a[...] * cc[...][:, :1], wls[i][...],
                        preferred_element_type=jnp.float32)
    out[...] = _ln_block(o, g[...], b[...]) + h


def _merge_small(aggs, cnts, h, wls, wr, bias, g, b, blk):
    m = h.shape[0]
    nagg = len(aggs)
    grid = (m // blk,)
    specA = pl.BlockSpec((blk, HID), lambda i: (i, 0))
    specC = pl.BlockSpec((blk, 16), lambda i: (i, 0))
    specW = pl.BlockSpec((HID, HID), lambda i: (0, 0))
    specV = pl.BlockSpec((1, HID), lambda i: (0, 0))
    in_specs = []
    args = []
    for a, cc in zip(aggs, cnts):
        in_specs += [specA, specC]
        args += [a, cc]
    in_specs.append(specA); args.append(h)
    for wl in wls:
        in_specs.append(specW); args.append(wl)
    in_specs += [specW, specV, specV, specV]
    args += [wr, bias.reshape(1, HID), g.reshape(1, HID), b.reshape(1, HID)]

    def body(*refs):
        _merge_small_kernel(nagg, refs)

    return pl.pallas_call(
        body, grid=grid, in_specs=in_specs, out_specs=specA,
        out_shape=_f32((m, HID)),
    )(*args)


def _classifier_kernel(x_ref, w_ref, b_ref, o_ref):
    o_ref[...] = (jnp.dot(x_ref[...], w_ref[...],
                          preferred_element_type=jnp.float32) + b_ref[...])


def _classifier(h_pad, wt, b):
    blk = 1000
    grid = (N_OCC // blk,)
    return pl.pallas_call(
        _classifier_kernel,
        grid=grid,
        in_specs=[
            pl.BlockSpec((blk, HID), lambda i: (i, 0)),
            pl.BlockSpec((HID, NUM_CLASSES), lambda i: (0, 0)),
            pl.BlockSpec((1, NUM_CLASSES), lambda i: (0, 0)),
        ],
        out_specs=pl.BlockSpec((blk, NUM_CLASSES), lambda i: (i, 0)),
        out_shape=_f32((N_OCC, NUM_CLASSES)),
    )(h_pad, wt, b.reshape(1, NUM_CLASSES))


def _pad_rows(x, n):
    return jnp.pad(x, ((0, n - x.shape[0]), (0, 0)))


def _pad_idx(idx, n, mode, npad_dst=0):
    e = idx.shape[0]
    i = jnp.arange(n - e, dtype=jnp.int32)
    if mode == "src":
        fill = (i * 97) % jnp.int32(1024)
    else:
        fill = npad_dst + (i % jnp.int32(1024 if npad_dst == N_OCC else 120))
    return jnp.concatenate([idx.astype(jnp.int32), fill])


def kernel(x_occ, x_chord, x_sec, params, next_src, next_dst, prev_src,
           prev_dst, inst_src, inst_dst, insec_src, insec_dst, nsec_src,
           nsec_dst):
    Wo, bo = params["occ_proj"]
    Wc, bc = params["chord_proj"]
    Ws, bs = params["sec_proj"]

    xo = _pad_rows(x_occ, NP_OCC)
    xc = _pad_rows(x_chord, NP_CH)
    xs = _pad_rows(x_sec, NP_SEC)

    n_s = _pad_idx(next_src, EPAD, "src")
    n_d = _pad_idx(next_dst, EPAD, "dst", N_OCC)
    p_s = _pad_idx(prev_src, EPAD, "src")
    p_d = _pad_idx(prev_dst, EPAD, "dst", N_OCC)
    ir_s = _pad_idx(inst_dst, EPAD, "src")
    ir_d = _pad_idx(inst_src, EPAD, "dst", N_OCC)
    sr_s = _pad_idx(insec_dst, EPAD, "src")
    sr_d = _pad_idx(insec_src, EPAD, "dst", N_OCC)
    io_s = _pad_idx(inst_src, EPAD, "src")
    io_d = _pad_idx(inst_dst, EPAD, "dst", N_CH)
    is_s = _pad_idx(insec_src, EPAD, "src")
    is_d = _pad_idx(insec_dst, EPAD, "dst", N_SEC)
    ns_s = _pad_idx(nsec_src, EPAD_NS, "src")
    ns_d = _pad_idx(nsec_dst, EPAD_NS, "dst", N_SEC)
    idx = (n_s, n_d, p_s, p_d, ir_s, ir_d, sr_s, sr_d,
           io_s, io_d, is_s, is_d, ns_s, ns_d)

    c_n, c_p, c_ir, c_sr, c_io, c_is, c_ns = _sc_counts(
        n_d, p_d, ir_d, sr_d, io_d, is_d, ns_d)
    c_n = _inv(c_n, 0.25, 1024)
    c_p = _inv(c_p, 0.25, 1024)
    c_ir = _inv(c_ir, 0.25, 1024)
    c_sr = _inv(c_sr, 0.25, 1024)
    c_io = _inv(c_io, 1.0, 1024)
    c_is = _inv(c_is, 1.0, 1280)
    c_ns = _inv(c_ns, 1.0, 1280)

    tcf, = _mm(xc, [Wo[:, 64:].T], [jnp.zeros((HID,), jnp.float32)], blk=1024)
    cfh = _sc_cf(ir_d, io_d, tcf)
    h_occ, = _mm(xo, [Wo[:, :64].T], [bo], add=cfh, blk=1024)
    h_ch, = _mm(xc, [Wc.T], [bc], blk=1024)
    h_sec, = _mm(xs, [Ws.T], [bs], blk=1280)

    zb = jnp.zeros((HID,), jnp.float32)
    for l in range(NUM_LAYERS):
        p = params["layers"][l]
        g, b = params["norms"][l]
        Wn_l, bn, Wn_r = p["next"]
        Wp_l, bp, Wp_r = p["prev"]
        Wir_l, bir, Wir_r = p["inst_rev"]
        Wsr_l, bsr, Wsr_r = p["sec_rev"]
        Wio_l, bio, Wio_r = p["instance_of"]
        Wis_l, bis, Wis_r = p["in_section"]
        Wns_l, bns, Wns_r = p["next_section"]

        wr_sum = 0.25 * (Wn_r + Wp_r + Wir_r + Wsr_r).T
        bsum = 0.25 * (bn + bp + bir + bsr)

        Tn, Tp, R = _mm(h_occ, [Wn_l.T, Wp_l.T, wr_sum], [zb, zb, bsum],
                        blk=1024)
        Tir, = _mm(h_ch, [Wir_l.T], [zb], blk=1024)
        Tsr, = _mm(h_sec, [Wsr_l.T], [zb], blk=1280)

        A_n, A_p, A_ir, A_sr, A_io, A_is, A_ns = _sc_layer(
            Tn, Tp, Tir, Tsr, h_occ, h_sec, idx)

        h_occ = _merge_occ((A_n, A_p, A_ir, A_sr), (c_n, c_p, c_ir, c_sr),
                           R, h_occ, g, b, jnp.zeros((HID,), jnp.float32))
        h_ch = _merge_small([A_io], [c_io], h_ch, [Wio_l.T], Wio_r.T, bio,
                            g, b, blk=1024)
        h_sec = _merge_small([A_is, A_ns], [c_is, c_ns], h_sec,
                             [0.5 * Wis_l.T, 0.5 * Wns_l.T],
                             0.5 * (Wis_r + Wns_r).T, 0.5 * (bis + bns),
                             g, b, blk=1280)

    Wcl, bcl = params["classifier"]
    return _classifier(h_occ, Wcl.T, bcl)
